# baseline (speedup 1.0000x reference)
import jax
import jax.numpy as jnp
from jax import lax
from jax.experimental import pallas as pl
from jax.experimental.pallas import tpu as pltpu
from jax.experimental.pallas import tpu_sc as plsc
import functools

MARGIN = 0.5

NC = 2
NS = 16
L = 16
NW = NC * NS

B = 16384
D = 128
PER_W = B // NW
CH = 128
NCHUNK = PER_W // CH
DJ = D // L

_mesh = plsc.VectorSubcoreMesh(
    core_axis_name="c", subcore_axis_name="s", num_cores=NC, num_subcores=NS)


_SCRATCH = [
    pltpu.VMEM((CH, D), jnp.float32),
    pltpu.VMEM((CH, D), jnp.float32),
    pltpu.VMEM((CH, D), jnp.float32),
    pltpu.VMEM((CH, D), jnp.float32),
    pltpu.VMEM((CH, D), jnp.float32),
    pltpu.VMEM((CH, D), jnp.float32),
    pltpu.VMEM((NCHUNK, CH), jnp.int32),
    pltpu.VMEM((NCHUNK, CH), jnp.int32),
    pltpu.VMEM((NCHUNK, CH), jnp.int32),
    pltpu.VMEM((L,), jnp.float32),
    pltpu.SemaphoreType.DMA,
    pltpu.SemaphoreType.DMA,
]


_SCHED = ((0, 0, 8), (0, 8, 24), (0, 32, 96), (1, 0, CH), (2, 0, CH),
          (3, 0, 96), (3, 96, 24), (3, 120, 8))


def _body(aidx_hbm, pidx_hbm, nidx_hbm, emb_hbm, out_hbm,
                arows0, prows0, nrows0, arows1, prows1, nrows1,
                aidx_v, pidx_v, nidx_v, accv, sem0, sem1):
    cid = lax.axis_index("c")
    sid = lax.axis_index("s")
    wid = sid * NC + cid

    lane = lax.iota(jnp.int32, L)
    zero = jnp.zeros((L,), jnp.float32)
    perms = [lane ^ sh for sh in (8, 4, 2, 1)]

    bufs = ((arows0, prows0, nrows0), (arows1, prows1, nrows1))
    sems = (sem0, sem1)

    def issue(k, b):
        r, col, n_ = _SCHED[k]
        a, p, n = bufs[b]
        return (
            pltpu.async_copy(
                emb_hbm.at[aidx_v.at[r, pl.ds(col, n_)]], a.at[pl.ds(0, n_)],
                sems[b]),
            pltpu.async_copy(
                emb_hbm.at[pidx_v.at[r, pl.ds(col, n_)]], p.at[pl.ds(0, n_)],
                sems[b]),
            pltpu.async_copy(
                emb_hbm.at[nidx_v.at[r, pl.ds(col, n_)]], n.at[pl.ds(0, n_)],
                sems[b]),
        )

    def make_trip_body(a_ref, p_ref, n_ref):
        def one_triplet(i):
            s0 = s1 = zero
            for j in range(DJ):
                va = a_ref[i, pl.ds(j * L, L)]
                vp = p_ref[i, pl.ds(j * L, L)]
                vn = n_ref[i, pl.ds(j * L, L)]
                d1 = va - vp
                d2 = va - vn
                if j % 2 == 0:
                    s0 = s0 + (d1 * d1 - d2 * d2)
                else:
                    s1 = s1 + (d1 * d1 - d2 * d2)
            s = s0 + s1
            for p in perms:
                s = s + s.at[p].get(mode="promise_in_bounds")
            return jnp.maximum(s + MARGIN, 0.0)

        def trip_body(i, acc):
            return acc + one_triplet(i)

        return trip_body

    pltpu.sync_copy(aidx_hbm.at[wid], aidx_v)
    pltpu.sync_copy(pidx_hbm.at[wid], pidx_v)
    pltpu.sync_copy(nidx_hbm.at[wid], nidx_v)
    descs = {0: issue(0, 0), 1: issue(1, 1)}
    acc = zero
    for k in range(len(_SCHED)):
        b = k % 2
        for dsc in descs.pop(k):
            dsc.wait()
        acc = lax.fori_loop(0, _SCHED[k][2], make_trip_body(*bufs[b]), acc)
        if k + 2 < len(_SCHED):
            descs[k + 2] = issue(k + 2, b)

    accv[...] = acc * (1.0 / (L * B))
    pltpu.sync_copy(accv, out_hbm.at[wid])


_triplet_sc = pl.kernel(
    _body,
    out_type=jax.ShapeDtypeStruct((NW, L), jnp.float32),
    mesh=_mesh,
    compiler_params=pltpu.CompilerParams(needs_layout_passes=False),
    scratch_types=_SCRATCH,
)


def kernel(triplets, embeddings):
    t = triplets.astype(jnp.int32)
    aidx = t[:, 0].reshape(NW, NCHUNK, CH)
    pidx = t[:, 1].reshape(NW, NCHUNK, CH)
    nidx = t[:, 2].reshape(NW, NCHUNK, CH)
    out = _triplet_sc(aidx, pidx, nidx, embeddings)
    return jnp.sum(out)


# speedup vs baseline: 1.0751x; 1.0751x over previous
import jax
import jax.numpy as jnp
from jax import lax
from jax.experimental import pallas as pl
from jax.experimental.pallas import tpu as pltpu
from jax.experimental.pallas import tpu_sc as plsc
import functools

MARGIN = 0.5

NC = 2
NS = 16
L = 16
NW = NC * NS

B = 16384
D = 128
PER_W = B // NW
CH = 128
NCHUNK = PER_W // CH
DJ = D // L

_mesh = plsc.VectorSubcoreMesh(
    core_axis_name="c", subcore_axis_name="s", num_cores=NC, num_subcores=NS)


_SCRATCH = [
    pltpu.VMEM((CH, D), jnp.float32),
    pltpu.VMEM((CH, D), jnp.float32),
    pltpu.VMEM((CH, D), jnp.float32),
    pltpu.VMEM((CH, D), jnp.float32),
    pltpu.VMEM((CH, D), jnp.float32),
    pltpu.VMEM((CH, D), jnp.float32),
    pltpu.VMEM((NCHUNK, CH), jnp.int32),
    pltpu.VMEM((NCHUNK, CH), jnp.int32),
    pltpu.VMEM((NCHUNK, CH), jnp.int32),
    pltpu.VMEM((L,), jnp.float32),
    pltpu.SemaphoreType.DMA,
    pltpu.SemaphoreType.DMA,
]


_SCHED = ((0, 0, 16), (0, 16, 112), (1, 0, CH), (2, 0, CH), (3, 0, CH))


def _body(aidx_hbm, pidx_hbm, nidx_hbm, emb_hbm, out_hbm,
                arows0, prows0, nrows0, arows1, prows1, nrows1,
                aidx_v, pidx_v, nidx_v, accv, sem0, sem1):
    cid = lax.axis_index("c")
    sid = lax.axis_index("s")
    wid = sid * NC + cid

    lane = lax.iota(jnp.int32, L)
    zero = jnp.zeros((L,), jnp.float32)
    perms = [lane ^ sh for sh in (8, 4, 2, 1)]

    bufs = ((arows0, prows0, nrows0), (arows1, prows1, nrows1))
    sems = (sem0, sem1)

    def issue(k, b):
        r, col, n_ = _SCHED[k]
        a, p, n = bufs[b]
        return (
            pltpu.async_copy(
                emb_hbm.at[aidx_v.at[r, pl.ds(col, n_)]], a.at[pl.ds(0, n_)],
                sems[b]),
            pltpu.async_copy(
                emb_hbm.at[pidx_v.at[r, pl.ds(col, n_)]], p.at[pl.ds(0, n_)],
                sems[b]),
            pltpu.async_copy(
                emb_hbm.at[nidx_v.at[r, pl.ds(col, n_)]], n.at[pl.ds(0, n_)],
                sems[b]),
        )

    def make_trip_body(a_ref, p_ref, n_ref):
        def one_triplet(i):
            s0 = s1 = zero
            for j in range(DJ):
                va = a_ref[i, pl.ds(j * L, L)]
                vp = p_ref[i, pl.ds(j * L, L)]
                vn = n_ref[i, pl.ds(j * L, L)]
                d1 = va - vp
                d2 = va - vn
                if j % 2 == 0:
                    s0 = s0 + (d1 * d1 - d2 * d2)
                else:
                    s1 = s1 + (d1 * d1 - d2 * d2)
            s = s0 + s1
            for p in perms:
                s = s + s.at[p].get(mode="promise_in_bounds")
            return jnp.maximum(s + MARGIN, 0.0)

        def trip_body(i, acc):
            return acc + one_triplet(i)

        return trip_body

    pltpu.sync_copy(aidx_hbm.at[wid], aidx_v)
    pltpu.sync_copy(pidx_hbm.at[wid], pidx_v)
    pltpu.sync_copy(nidx_hbm.at[wid], nidx_v)
    descs = {0: issue(0, 0), 1: issue(1, 1)}
    acc = zero
    for k in range(len(_SCHED)):
        b = k % 2
        for dsc in descs.pop(k):
            dsc.wait()
        acc = lax.fori_loop(0, _SCHED[k][2], make_trip_body(*bufs[b]), acc)
        if k + 2 < len(_SCHED):
            descs[k + 2] = issue(k + 2, b)

    accv[...] = acc * (1.0 / (L * B))
    pltpu.sync_copy(accv, out_hbm.at[wid])


_triplet_sc = pl.kernel(
    _body,
    out_type=jax.ShapeDtypeStruct((NW, L), jnp.float32),
    mesh=_mesh,
    compiler_params=pltpu.CompilerParams(needs_layout_passes=False),
    scratch_types=_SCRATCH,
)


def kernel(triplets, embeddings):
    t = triplets.astype(jnp.int32)
    aidx = t[:, 0].reshape(NW, NCHUNK, CH)
    pidx = t[:, 1].reshape(NW, NCHUNK, CH)
    nidx = t[:, 2].reshape(NW, NCHUNK, CH)
    out = _triplet_sc(aidx, pidx, nidx, embeddings)
    return jnp.sum(out)
